# baseline (speedup 1.0000x reference)
import functools
import jax
import jax.numpy as jnp
from jax import lax
from jax.experimental import pallas as pl
from jax.experimental.pallas import tpu as pltpu
from jax.experimental.pallas import tpu_sc as plsc

NC = 2
NS = 16
NW = NC * NS
L = 16

VOCAB = 100000
D = 64
SEQ = 200
BATCH = 4096

IDX_W = 100
KROWS = 4
CHUNK = KROWS * IDX_W
N_FLAT = BATCH * SEQ
ROWS_TOTAL = N_FLAT // IDX_W
CHUNKS_PER_W = ROWS_TOTAL // (KROWS * NW)


def _body(x_hbm, word_hbm, pos_hbm, out_hbm, pos_v, idx_v, gbuf, rows2, sem):
    wid = lax.axis_index("s") * NC + lax.axis_index("c")

    pltpu.sync_copy(pos_hbm, pos_v)

    def chunk_body(g, _):
        c = wid * CHUNKS_PER_W + g
        r0 = c * KROWS
        pltpu.sync_copy(x_hbm.at[pl.ds(r0, KROWS)], idx_v)
        cps = []
        for j in range(KROWS):
            cps.append(
                pltpu.async_copy(
                    word_hbm.at[idx_v.at[j]],
                    gbuf.at[pl.ds(j * IDX_W, IDX_W)],
                    sem,
                )
            )
        for cp in cps:
            cp.wait()

        def add_body(k, _):
            for p in range(2):
                pv = [pos_v[2 * k + p, pl.ds(d * L, L)] for d in range(4)]
                for h in range(2):
                    r = h * 200 + 2 * k + p
                    dr = h * 100 + k
                    for d in range(4):
                        rows2[dr, pl.ds(p * D + d * L, L)] = (
                            gbuf[r, pl.ds(d * L, L)] + pv[d]
                        )
            return ()

        lax.fori_loop(0, 100, add_body, ())

        pltpu.sync_copy(rows2, out_hbm.at[pl.ds(c * (CHUNK // 2), CHUNK // 2)])
        return ()

    lax.fori_loop(0, CHUNKS_PER_W, chunk_body, ())


@jax.jit
def kernel(x, word_table, pos_table):
    x_flat = x.reshape(ROWS_TOTAL, IDX_W)
    mesh = plsc.VectorSubcoreMesh(core_axis_name="c", subcore_axis_name="s")
    out = pl.kernel(
        _body,
        out_type=jax.ShapeDtypeStruct((N_FLAT // 2, 2 * D), jnp.float32),
        mesh=mesh,
        compiler_params=pltpu.CompilerParams(use_tc_tiling_on_sc=False),
        scratch_types=[
            pltpu.VMEM((SEQ, D), jnp.float32),
            pltpu.VMEM((KROWS, IDX_W), jnp.int32),
            pltpu.VMEM((CHUNK, D), jnp.float32),
            pltpu.VMEM((CHUNK // 2, 2 * D), jnp.float32),
            pltpu.SemaphoreType.DMA,
        ],
    )(x_flat, word_table, pos_table)
    return out.reshape(BATCH, SEQ, D)


# speedup vs baseline: 1.0043x; 1.0043x over previous
import functools
import jax
import jax.numpy as jnp
from jax import lax
from jax.experimental import pallas as pl
from jax.experimental.pallas import tpu as pltpu
from jax.experimental.pallas import tpu_sc as plsc

NC = 2
NS = 16
NW = NC * NS
L = 16

VOCAB = 100000
D = 64
SEQ = 200
BATCH = 4096

IDX_G = 80
CHUNK = 400
N_FLAT = BATCH * SEQ
CHUNKS = N_FLAT // CHUNK
CHUNKS_PER_W = CHUNKS // NW


def _body(x_hbm, word_hbm, pos_hbm, out_hbm, pos_v, idx_v, gbuf, sem):
    wid = lax.axis_index("s") * NC + lax.axis_index("c")

    pltpu.sync_copy(pos_hbm, pos_v)

    def chunk_body(g, _):
        c = wid * CHUNKS_PER_W + g
        pltpu.sync_copy(x_hbm.at[pl.ds(c * CHUNK, CHUNK)], idx_v)
        cps = []
        for j in range(CHUNK // IDX_G):
            cps.append(
                pltpu.async_copy(
                    word_hbm.at[idx_v.at[pl.ds(j * IDX_G, IDX_G)]],
                    gbuf.at[pl.ds(j * IDX_G, IDX_G)],
                    sem,
                )
            )
        for cp in cps:
            cp.wait()

        def add_body(s, _):
            for d in range(D // L):
                p = pos_v[pl.ds(s * D + d * L, L)]
                for h in range(CHUNK // SEQ):
                    r = h * SEQ + s
                    gbuf[r, pl.ds(d * L, L)] = gbuf[r, pl.ds(d * L, L)] + p
            return ()

        lax.fori_loop(0, SEQ, add_body, ())

        pltpu.sync_copy(gbuf, out_hbm.at[pl.ds(c * CHUNK, CHUNK)])
        return ()

    lax.fori_loop(0, CHUNKS_PER_W, chunk_body, ())


@jax.jit
def kernel(x, word_table, pos_table):
    x_flat = x.reshape(-1)
    pos_flat = pos_table.reshape(-1)
    mesh = plsc.VectorSubcoreMesh(core_axis_name="c", subcore_axis_name="s")
    out = pl.kernel(
        _body,
        out_type=jax.ShapeDtypeStruct((N_FLAT, D), jnp.float32),
        mesh=mesh,
        compiler_params=pltpu.CompilerParams(use_tc_tiling_on_sc=False),
        scratch_types=[
            pltpu.VMEM((SEQ * D,), jnp.float32),
            pltpu.VMEM((CHUNK,), jnp.int32),
            pltpu.VMEM((CHUNK, D), jnp.float32),
            pltpu.SemaphoreType.DMA,
        ],
    )(x_flat, word_table, pos_flat)
    return out.reshape(BATCH, SEQ, D)


# speedup vs baseline: 1.2985x; 1.2929x over previous
import functools
import jax
import jax.numpy as jnp
from jax import lax
from jax.experimental import pallas as pl
from jax.experimental.pallas import tpu as pltpu
from jax.experimental.pallas import tpu_sc as plsc

NC = 2
NS = 16
NW = NC * NS
L = 16

VOCAB = 100000
D = 64
SEQ = 200
BATCH = 4096

D_PER_W = D // NW
GROUPS = BATCH // L
G_IN = 8


def _splat(val):
    return jax.lax.broadcast_in_dim(val, (L,), ())


def _body(xt, wt, post, out, row_v, posr_v, ib0, ib1, ob0, ob1,
          si0, si1, so0, so1):
    w = lax.axis_index("s") * NC + lax.axis_index("c")
    ibufs, obufs, sis, sos = (ib0, ib1), (ob0, ob1), (si0, si1), (so0, so1)

    def out_slice(s, dt, di):
        return out.at[s, dt, pl.ds(0, 32), pl.ds(di, 1), pl.ds(0, 128)]

    def phase(dn, _):
        d = w * D_PER_W + dn
        dt = d // 8
        di = d % 8
        pltpu.sync_copy(wt.at[d], row_v)
        pltpu.sync_copy(post.at[d], posr_v)
        pltpu.async_copy(xt.at[0], ib0, si0)
        pltpu.async_copy(xt.at[1], ib1, si1)

        def sbody(k, _):
            for par in range(2):
                s = 2 * k + par
                ibuf, obuf = ibufs[par], obufs[par]
                pltpu.make_async_copy(xt.at[s], ibuf, sis[par]).wait()
                @pl.when(k > 0)
                def _():
                    pltpu.make_async_copy(
                        obuf, out_slice(s, dt, di), sos[par]
                    ).wait()

                pv = plsc.load_gather(posr_v, [_splat(s)])

                def gbody(go, _):
                    base = go * (G_IN * L)
                    for gi in range(G_IN):
                        off = base + gi * L
                        iv = ibuf[pl.ds(off, L)]
                        gv = plsc.load_gather(row_v, [iv])
                        bt = base // 128
                        bo = (base % 128) + gi * L
                        obuf[bt, 0, pl.ds(bo, L)] = gv + pv
                    return ()

                lax.fori_loop(0, GROUPS // G_IN, gbody, ())
                pltpu.async_copy(obuf, out_slice(s, dt, di), sos[par])
                @pl.when(s + 2 < SEQ)
                def _():
                    pltpu.async_copy(xt.at[s + 2], ibuf, sis[par])

            return ()

        lax.fori_loop(0, SEQ // 2, sbody, ())
        for par in range(2):
            pltpu.make_async_copy(
                obufs[par], out_slice(0, 0, 0), sos[par]
            ).wait()
        return ()

    lax.fori_loop(0, D_PER_W, phase, ())


@jax.jit
def kernel(x, word_table, pos_table):
    xt = x.T
    wt = word_table.T
    post = pos_table.T
    mesh = plsc.VectorSubcoreMesh(core_axis_name="c", subcore_axis_name="s")
    out5 = pl.kernel(
        _body,
        out_type=jax.ShapeDtypeStruct((SEQ, 8, 32, 8, 128), jnp.float32),
        mesh=mesh,
        compiler_params=pltpu.CompilerParams(
            use_tc_tiling_on_sc=False, needs_layout_passes=False
        ),
        scratch_types=[
            pltpu.VMEM((VOCAB,), jnp.float32),
            pltpu.VMEM((SEQ,), jnp.float32),
            pltpu.VMEM((BATCH,), jnp.int32),
            pltpu.VMEM((BATCH,), jnp.int32),
            pltpu.VMEM((32, 1, 128), jnp.float32),
            pltpu.VMEM((32, 1, 128), jnp.float32),
            pltpu.SemaphoreType.DMA,
            pltpu.SemaphoreType.DMA,
            pltpu.SemaphoreType.DMA,
            pltpu.SemaphoreType.DMA,
        ],
    )(xt, wt, post)
    return out5.transpose(2, 4, 0, 1, 3).reshape(BATCH, SEQ, D)


# speedup vs baseline: 2.3015x; 1.7723x over previous
import functools
import jax
import jax.numpy as jnp
from jax import lax
from jax.experimental import pallas as pl
from jax.experimental.pallas import tpu as pltpu
from jax.experimental.pallas import tpu_sc as plsc

NC = 2
NS = 16
NW = NC * NS
L = 16

VOCAB = 100000
D = 64
SEQ = 200
BATCH = 4096

D_PER_W = D // NW
GROUPS = BATCH // L
G_IN = 8


def _splat(val):
    return jax.lax.broadcast_in_dim(val, (L,), ())


def _body(xt, wt, post, out, row_v, posr_v, ib0, ib1, ob0, ob1,
          si0, si1, so0, so1):
    w = lax.axis_index("s") * NC + lax.axis_index("c")
    ibufs, obufs, sis, sos = (ib0, ib1), (ob0, ob1), (si0, si1), (so0, so1)

    def out_slice(s, dt, di):
        return out.at[s, dt, pl.ds(0, 32), pl.ds(di, 1), pl.ds(0, 128)]

    def phase(dn, _):
        d = w * D_PER_W + dn
        dt = d // 8
        di = d % 8
        pltpu.sync_copy(wt.at[d], row_v)
        pltpu.sync_copy(post.at[d], posr_v)
        pltpu.async_copy(xt.at[0], ib0, si0)
        pltpu.async_copy(xt.at[1], ib1, si1)

        def sbody(k, _):
            for par in range(2):
                s = 2 * k + par
                ibuf, obuf = ibufs[par], obufs[par]
                pltpu.make_async_copy(xt.at[s], ibuf, sis[par]).wait()
                @pl.when(k > 0)
                def _():
                    pltpu.make_async_copy(
                        obuf, out_slice(s, dt, di), sos[par]
                    ).wait()

                pv = plsc.load_gather(posr_v, [_splat(s)])

                @plsc.parallel_loop(0, GROUPS // G_IN, 1, unroll=4)
                def _(go):
                    base = go * (G_IN * L)
                    for gi in range(G_IN):
                        iv = ibuf[pl.ds(base + gi * L, L)]
                        gv = plsc.load_gather(row_v, [iv])
                        obuf[go, 0, pl.ds(gi * L, L)] = gv + pv
                pltpu.async_copy(obuf, out_slice(s, dt, di), sos[par])
                @pl.when(s + 2 < SEQ)
                def _():
                    pltpu.async_copy(xt.at[s + 2], ibuf, sis[par])

            return ()

        lax.fori_loop(0, SEQ // 2, sbody, ())
        for par in range(2):
            pltpu.make_async_copy(
                obufs[par], out_slice(0, 0, 0), sos[par]
            ).wait()
        return ()

    lax.fori_loop(0, D_PER_W, phase, ())


@jax.jit
def kernel(x, word_table, pos_table):
    xt = x.T
    wt = word_table.T
    post = pos_table.T
    mesh = plsc.VectorSubcoreMesh(core_axis_name="c", subcore_axis_name="s")
    out5 = pl.kernel(
        _body,
        out_type=jax.ShapeDtypeStruct((SEQ, 8, 32, 8, 128), jnp.float32),
        mesh=mesh,
        compiler_params=pltpu.CompilerParams(
            use_tc_tiling_on_sc=False, needs_layout_passes=False
        ),
        scratch_types=[
            pltpu.VMEM((VOCAB,), jnp.float32),
            pltpu.VMEM((SEQ,), jnp.float32),
            pltpu.VMEM((BATCH,), jnp.int32),
            pltpu.VMEM((BATCH,), jnp.int32),
            pltpu.VMEM((32, 1, 128), jnp.float32),
            pltpu.VMEM((32, 1, 128), jnp.float32),
            pltpu.SemaphoreType.DMA,
            pltpu.SemaphoreType.DMA,
            pltpu.SemaphoreType.DMA,
            pltpu.SemaphoreType.DMA,
        ],
    )(xt, wt, post)
    return out5.transpose(2, 4, 0, 1, 3).reshape(BATCH, SEQ, D)
